# baseline (speedup 1.0000x reference)
import jax
import jax.numpy as jnp
from jax import lax
from jax.experimental import pallas as pl
from jax.experimental.pallas import tpu as pltpu
from jax.experimental.pallas import tpu_sc as plsc

B, S, D = 4, 2048, 768
EPS = 1e-12

_info = plsc.get_sparse_core_info()
NC, NS = _info.num_cores, _info.num_subcores
NW = NC * NS
TOK_PER_W = S // NW


def _sc_gather(word_hbm, ids_hbm, out_hbm, idx_v, buf, gsem, ssem):
    wid = lax.axis_index("s") * NC + lax.axis_index("c")
    pltpu.sync_copy(ids_hbm.at[wid], idx_v)
    pltpu.async_copy(word_hbm.at[idx_v.at[0]], buf, gsem).wait()
    pltpu.async_copy(buf, out_hbm.at[pl.ds(wid * TOK_PER_W, TOK_PER_W)],
                     ssem).wait()


def _gather_rows(word_embeddings, ids):
    ids3 = ids.reshape(NW, 1, TOK_PER_W)
    mesh = plsc.VectorSubcoreMesh(core_axis_name="c", subcore_axis_name="s")
    return pl.kernel(
        _sc_gather,
        mesh=mesh,
        out_type=jax.ShapeDtypeStruct((S, D), jnp.float32),
        scratch_types=[
            pltpu.VMEM((1, TOK_PER_W), jnp.int32),
            pltpu.VMEM((TOK_PER_W, D), jnp.float32),
            pltpu.SemaphoreType.DMA,
            pltpu.SemaphoreType.DMA,
        ],
    )(word_embeddings, ids3)


ROWS_BLK = 256
POS_BLKS = S // ROWS_BLK


def _tc_epilogue(g_ref, p_ref, tt_tab_ref, tt_ref, gamma_ref, beta_ref, o_ref):
    x = g_ref[...] + p_ref[...]
    tt = tt_ref[0].astype(jnp.float32)
    row0 = tt_tab_ref[0:1, :]
    row1 = tt_tab_ref[1:2, :]
    x = x + row0 + tt * (row1 - row0)
    mean = jnp.mean(x, axis=-1, keepdims=True)
    d = x - mean
    var = jnp.mean(d * d, axis=-1, keepdims=True)
    o_ref[...] = d * lax.rsqrt(var + EPS) * gamma_ref[...] + beta_ref[...]


def _epilogue(gathered, position_embeddings, token_type_embeddings, tt_ids,
              ln_gamma, ln_beta):
    tt3 = tt_ids.reshape(POS_BLKS, ROWS_BLK, 1)
    return pl.pallas_call(
        _tc_epilogue,
        grid=(POS_BLKS,),
        in_specs=[
            pl.BlockSpec((ROWS_BLK, D), lambda p: (p, 0)),
            pl.BlockSpec((ROWS_BLK, D), lambda p: (p, 0)),
            pl.BlockSpec((2, D), lambda p: (0, 0)),
            pl.BlockSpec((1, ROWS_BLK, 1), lambda p: (p, 0, 0)),
            pl.BlockSpec((1, D), lambda p: (0, 0)),
            pl.BlockSpec((1, D), lambda p: (0, 0)),
        ],
        out_specs=pl.BlockSpec((ROWS_BLK, D), lambda p: (p, 0)),
        out_shape=jax.ShapeDtypeStruct((S, D), jnp.float32),
    )(gathered, position_embeddings, token_type_embeddings, tt3,
      ln_gamma.reshape(1, D), ln_beta.reshape(1, D))


def kernel(input_ids, token_type_ids, word_embeddings, position_embeddings,
           token_type_embeddings, ln_gamma, ln_beta):
    ids = input_ids.astype(jnp.int32)
    tt_ids = token_type_ids.astype(jnp.int32)
    outs = []
    for b in range(B):
        g = _gather_rows(word_embeddings, ids[b])
        outs.append(_epilogue(g, position_embeddings, token_type_embeddings,
                              tt_ids[b], ln_gamma, ln_beta))
    return jnp.stack(outs, axis=0)


# speedup vs baseline: 1.0013x; 1.0013x over previous
import jax
import jax.numpy as jnp
from jax import lax
from jax.experimental import pallas as pl
from jax.experimental.pallas import tpu as pltpu
from jax.experimental.pallas import tpu_sc as plsc

B, S, D = 4, 2048, 768
EPS = 1e-12

_info = plsc.get_sparse_core_info()
NC, NS = _info.num_cores, _info.num_subcores
NW = NC * NS
TOK_PER_W = S // NW


def _sc_gather(word_hbm, ids_hbm, out_hbm, idx_v, buf, gsem, ssem):
    wid = lax.axis_index("s") * NC + lax.axis_index("c")
    pltpu.sync_copy(ids_hbm.at[wid], idx_v)
    pltpu.async_copy(word_hbm.at[idx_v.at[0]], buf, gsem).wait()
    pltpu.async_copy(buf, out_hbm.at[pl.ds(wid * TOK_PER_W, TOK_PER_W)],
                     ssem).wait()


def _gather_rows(word_embeddings, ids):
    ids3 = ids.reshape(NW, 1, TOK_PER_W)
    mesh = plsc.VectorSubcoreMesh(core_axis_name="c", subcore_axis_name="s")
    return pl.kernel(
        _sc_gather,
        mesh=mesh,
        out_type=jax.ShapeDtypeStruct((S, D), jnp.float32),
        scratch_types=[
            pltpu.VMEM((1, TOK_PER_W), jnp.int32),
            pltpu.VMEM((TOK_PER_W, D), jnp.float32),
            pltpu.SemaphoreType.DMA,
            pltpu.SemaphoreType.DMA,
        ],
    )(word_embeddings, ids3)


ROWS_BLK = 256
POS_BLKS = S // ROWS_BLK


def _tc_epilogue(g_ref, p_ref, tt_tab_ref, tt_ref, gamma_ref, beta_ref, o_ref):
    x = g_ref[...] + p_ref[...]
    tt = tt_ref[0].astype(jnp.float32)
    row0 = tt_tab_ref[0:1, :]
    row1 = tt_tab_ref[1:2, :]
    x = x + row0 + tt * (row1 - row0)
    mean = jnp.mean(x, axis=-1, keepdims=True)
    d = x - mean
    var = jnp.mean(d * d, axis=-1, keepdims=True)
    o_ref[...] = d * lax.rsqrt(var + EPS) * gamma_ref[...] + beta_ref[...]


def _epilogue(gathered, position_embeddings, token_type_embeddings, tt_ids,
              ln_gamma, ln_beta):
    tt3 = tt_ids.reshape(POS_BLKS, ROWS_BLK, 1)
    return pl.pallas_call(
        _tc_epilogue,
        grid=(POS_BLKS,),
        in_specs=[
            pl.BlockSpec((ROWS_BLK, D), lambda p: (p, 0)),
            pl.BlockSpec((ROWS_BLK, D), lambda p: (p, 0)),
            pl.BlockSpec((2, D), lambda p: (0, 0)),
            pl.BlockSpec((1, ROWS_BLK, 1), lambda p: (p, 0, 0)),
            pl.BlockSpec((1, D), lambda p: (0, 0)),
            pl.BlockSpec((1, D), lambda p: (0, 0)),
        ],
        out_specs=pl.BlockSpec((ROWS_BLK, D), lambda p: (p, 0)),
        out_shape=jax.ShapeDtypeStruct((S, D), jnp.float32),
    )(gathered, position_embeddings, token_type_embeddings, tt3,
      ln_gamma.reshape(1, D), ln_beta.reshape(1, D))


def kernel(input_ids, token_type_ids, word_embeddings, position_embeddings,
           token_type_embeddings, ln_gamma, ln_beta):
    ids = input_ids.astype(jnp.int32)
    tt_ids = token_type_ids.astype(jnp.int32)
    gs = [_gather_rows(word_embeddings, ids[b]) for b in range(B)]
    outs = [_epilogue(g, position_embeddings, token_type_embeddings,
                      tt_ids[b], ln_gamma, ln_beta) for b, g in enumerate(gs)]
    return jnp.stack(outs, axis=0)


# speedup vs baseline: 1.4713x; 1.4694x over previous
import jax
import jax.numpy as jnp
from jax import lax
from jax.experimental import pallas as pl
from jax.experimental.pallas import tpu as pltpu
from jax.experimental.pallas import tpu_sc as plsc

B, S, D = 4, 2048, 768
N = B * S
EPS = 1e-12

_info = plsc.get_sparse_core_info()
NC, NS = _info.num_cores, _info.num_subcores
NW = NC * NS
TOK_PER_W = N // NW
CHUNK = 64
NCHUNK = TOK_PER_W // CHUNK


def _sc_gather(word_hbm, ids_hbm, out_hbm, idx_v, buf0, buf1, gs0, gs1, ss0,
               ss1):
    wid = lax.axis_index("s") * NC + lax.axis_index("c")
    pltpu.sync_copy(ids_hbm.at[wid], idx_v)
    base = wid * TOK_PER_W
    bufs = (buf0, buf1)
    gsems = (gs0, gs1)
    ssems = (ss0, ss1)
    gathers = [None] * NCHUNK
    scatters = [None] * NCHUNK
    gathers[0] = pltpu.async_copy(word_hbm.at[idx_v.at[0]], bufs[0], gsems[0])
    for j in range(NCHUNK):
        b = j % 2
        if j + 1 < NCHUNK:
            if j - 1 >= 0:
                scatters[j - 1].wait()
            gathers[j + 1] = pltpu.async_copy(
                word_hbm.at[idx_v.at[j + 1]], bufs[1 - b], gsems[1 - b])
        gathers[j].wait()
        scatters[j] = pltpu.async_copy(
            bufs[b], out_hbm.at[pl.ds(base + j * CHUNK, CHUNK)], ssems[b])
    scatters[NCHUNK - 2].wait()
    scatters[NCHUNK - 1].wait()


def _gather_rows(word_embeddings, ids):
    ids3 = ids.reshape(NW, NCHUNK, CHUNK)
    mesh = plsc.VectorSubcoreMesh(core_axis_name="c", subcore_axis_name="s")
    return pl.kernel(
        _sc_gather,
        mesh=mesh,
        out_type=jax.ShapeDtypeStruct((N, D), jnp.float32),
        scratch_types=[
            pltpu.VMEM((NCHUNK, CHUNK), jnp.int32),
            pltpu.VMEM((CHUNK, D), jnp.float32),
            pltpu.VMEM((CHUNK, D), jnp.float32),
            pltpu.SemaphoreType.DMA,
            pltpu.SemaphoreType.DMA,
            pltpu.SemaphoreType.DMA,
            pltpu.SemaphoreType.DMA,
        ],
    )(word_embeddings, ids3)


ROWS_BLK = 512
POS_BLKS = S // ROWS_BLK


def _tc_epilogue(g_ref, p_ref, tt_tab_ref, tt_ref, gamma_ref, beta_ref, o_ref):
    x = g_ref[...] + p_ref[...]
    tt = tt_ref[0].astype(jnp.float32)
    row0 = tt_tab_ref[0:1, :]
    row1 = tt_tab_ref[1:2, :]
    x = x + row0 + tt * (row1 - row0)
    mean = jnp.mean(x, axis=-1, keepdims=True)
    d = x - mean
    var = jnp.mean(d * d, axis=-1, keepdims=True)
    o_ref[...] = d * lax.rsqrt(var + EPS) * gamma_ref[...] + beta_ref[...]


def _epilogue(gathered, position_embeddings, token_type_embeddings, tt_ids,
              ln_gamma, ln_beta):
    tt3 = tt_ids.reshape(N // ROWS_BLK, ROWS_BLK, 1)
    return pl.pallas_call(
        _tc_epilogue,
        grid=(POS_BLKS, B),
        in_specs=[
            pl.BlockSpec((ROWS_BLK, D), lambda p, b: (b * POS_BLKS + p, 0)),
            pl.BlockSpec((ROWS_BLK, D), lambda p, b: (p, 0)),
            pl.BlockSpec((2, D), lambda p, b: (0, 0)),
            pl.BlockSpec((1, ROWS_BLK, 1), lambda p, b: (b * POS_BLKS + p, 0, 0)),
            pl.BlockSpec((1, D), lambda p, b: (0, 0)),
            pl.BlockSpec((1, D), lambda p, b: (0, 0)),
        ],
        out_specs=pl.BlockSpec((ROWS_BLK, D), lambda p, b: (b * POS_BLKS + p, 0)),
        out_shape=jax.ShapeDtypeStruct((N, D), jnp.float32),
    )(gathered, position_embeddings, token_type_embeddings, tt3,
      ln_gamma.reshape(1, D), ln_beta.reshape(1, D))


def kernel(input_ids, token_type_ids, word_embeddings, position_embeddings,
           token_type_embeddings, ln_gamma, ln_beta):
    ids = input_ids.astype(jnp.int32)
    tt_ids = token_type_ids.astype(jnp.int32)
    g = _gather_rows(word_embeddings, ids.reshape(N))
    out = _epilogue(g, position_embeddings, token_type_embeddings,
                    tt_ids.reshape(N), ln_gamma, ln_beta)
    return out.reshape(B, S, D)


# speedup vs baseline: 1.5519x; 1.0548x over previous
import jax
import jax.numpy as jnp
from jax import lax
from jax.experimental import pallas as pl
from jax.experimental.pallas import tpu as pltpu
from jax.experimental.pallas import tpu_sc as plsc

B, S, D = 4, 2048, 768
N = B * S
EPS = 1e-12

_info = plsc.get_sparse_core_info()
NC, NS = _info.num_cores, _info.num_subcores
NW = NC * NS
TOK_PER_W = N // NW
CHUNK = 64
NCHUNK = TOK_PER_W // CHUNK


def _sc_gather(word_hbm, ids_hbm, out_hbm, idx_v, buf0, buf1, gs0, gs1, ss0,
               ss1):
    wid = lax.axis_index("s") * NC + lax.axis_index("c")
    pltpu.sync_copy(ids_hbm.at[wid], idx_v)
    base = wid * TOK_PER_W
    bufs = (buf0, buf1)
    gsems = (gs0, gs1)
    ssems = (ss0, ss1)
    gathers = [None] * NCHUNK
    scatters = [None] * NCHUNK
    gathers[0] = pltpu.async_copy(word_hbm.at[idx_v.at[0]], bufs[0], gsems[0])
    for j in range(NCHUNK):
        b = j % 2
        if j + 1 < NCHUNK:
            if j - 1 >= 0:
                scatters[j - 1].wait()
            gathers[j + 1] = pltpu.async_copy(
                word_hbm.at[idx_v.at[j + 1]], bufs[1 - b], gsems[1 - b])
        gathers[j].wait()
        scatters[j] = pltpu.async_copy(
            bufs[b], out_hbm.at[pl.ds(base + j * CHUNK, CHUNK)], ssems[b])
    scatters[NCHUNK - 2].wait()
    scatters[NCHUNK - 1].wait()


def _gather_rows(word_embeddings, ids):
    ids3 = ids.reshape(NW, NCHUNK, CHUNK)
    mesh = plsc.VectorSubcoreMesh(core_axis_name="c", subcore_axis_name="s")
    return pl.kernel(
        _sc_gather,
        mesh=mesh,
        out_type=jax.ShapeDtypeStruct((N, D), jnp.float32),
        scratch_types=[
            pltpu.VMEM((NCHUNK, CHUNK), jnp.int32),
            pltpu.VMEM((CHUNK, D), jnp.float32),
            pltpu.VMEM((CHUNK, D), jnp.float32),
            pltpu.SemaphoreType.DMA,
            pltpu.SemaphoreType.DMA,
            pltpu.SemaphoreType.DMA,
            pltpu.SemaphoreType.DMA,
        ],
    )(word_embeddings, ids3)


ROWS_BLK = 1024
POS_BLKS = S // ROWS_BLK


def _tc_epilogue(g_ref, p_ref, tt_tab_ref, tt_ref, gamma_ref, beta_ref, o_ref):
    x = g_ref[...] + p_ref[...]
    tt = tt_ref[0].astype(jnp.float32)
    row0 = tt_tab_ref[0:1, :]
    row1 = tt_tab_ref[1:2, :]
    x = x + row0 + tt * (row1 - row0)
    mean = jnp.mean(x, axis=-1, keepdims=True)
    d = x - mean
    var = jnp.mean(d * d, axis=-1, keepdims=True)
    o_ref[...] = d * lax.rsqrt(var + EPS) * gamma_ref[...] + beta_ref[...]


def _epilogue(gathered, position_embeddings, token_type_embeddings, tt_ids,
              ln_gamma, ln_beta):
    tt3 = tt_ids.reshape(N // ROWS_BLK, ROWS_BLK, 1)
    return pl.pallas_call(
        _tc_epilogue,
        grid=(POS_BLKS, B),
        in_specs=[
            pl.BlockSpec((ROWS_BLK, D), lambda p, b: (b * POS_BLKS + p, 0)),
            pl.BlockSpec((ROWS_BLK, D), lambda p, b: (p, 0)),
            pl.BlockSpec((2, D), lambda p, b: (0, 0)),
            pl.BlockSpec((1, ROWS_BLK, 1), lambda p, b: (b * POS_BLKS + p, 0, 0)),
            pl.BlockSpec((1, D), lambda p, b: (0, 0)),
            pl.BlockSpec((1, D), lambda p, b: (0, 0)),
        ],
        out_specs=pl.BlockSpec((ROWS_BLK, D), lambda p, b: (b * POS_BLKS + p, 0)),
        out_shape=jax.ShapeDtypeStruct((N, D), jnp.float32),
    )(gathered, position_embeddings, token_type_embeddings, tt3,
      ln_gamma.reshape(1, D), ln_beta.reshape(1, D))


def kernel(input_ids, token_type_ids, word_embeddings, position_embeddings,
           token_type_embeddings, ln_gamma, ln_beta):
    ids = input_ids.astype(jnp.int32)
    tt_ids = token_type_ids.astype(jnp.int32)
    g = _gather_rows(word_embeddings, ids.reshape(N))
    out = _epilogue(g, position_embeddings, token_type_embeddings,
                    tt_ids.reshape(N), ln_gamma, ln_beta)
    return out.reshape(B, S, D)


# speedup vs baseline: 1.5967x; 1.0289x over previous
import jax
import jax.numpy as jnp
from jax import lax
from jax.experimental import pallas as pl
from jax.experimental.pallas import tpu as pltpu
from jax.experimental.pallas import tpu_sc as plsc

B, S, D = 4, 2048, 768
N = B * S
EPS = 1e-12

_info = plsc.get_sparse_core_info()
NC, NS = _info.num_cores, _info.num_subcores
NW = NC * NS
TOK_PER_W = N // NW
CHUNK = 64
NCHUNK = TOK_PER_W // CHUNK


def _sc_gather(word_hbm, ids_hbm, out_hbm, idx_v, buf0, buf1, gs0, gs1, ss0,
               ss1):
    wid = lax.axis_index("s") * NC + lax.axis_index("c")
    pltpu.sync_copy(ids_hbm.at[wid], idx_v)
    base = wid * TOK_PER_W
    bufs = (buf0, buf1)
    gsems = (gs0, gs1)
    ssems = (ss0, ss1)
    gathers = [None] * NCHUNK
    scatters = [None] * NCHUNK
    gathers[0] = pltpu.async_copy(word_hbm.at[idx_v.at[0]], bufs[0], gsems[0])
    for j in range(NCHUNK):
        b = j % 2
        if j + 1 < NCHUNK:
            if j - 1 >= 0:
                scatters[j - 1].wait()
            gathers[j + 1] = pltpu.async_copy(
                word_hbm.at[idx_v.at[j + 1]], bufs[1 - b], gsems[1 - b])
        gathers[j].wait()
        scatters[j] = pltpu.async_copy(
            bufs[b], out_hbm.at[pl.ds(base + j * CHUNK, CHUNK)], ssems[b])
    scatters[NCHUNK - 2].wait()
    scatters[NCHUNK - 1].wait()


def _gather_rows(word_embeddings, ids):
    ids3 = ids.reshape(NW, NCHUNK, CHUNK)
    mesh = plsc.VectorSubcoreMesh(core_axis_name="c", subcore_axis_name="s")
    return pl.kernel(
        _sc_gather,
        mesh=mesh,
        out_type=jax.ShapeDtypeStruct((N, D), jnp.float32),
        scratch_types=[
            pltpu.VMEM((NCHUNK, CHUNK), jnp.int32),
            pltpu.VMEM((CHUNK, D), jnp.float32),
            pltpu.VMEM((CHUNK, D), jnp.float32),
            pltpu.SemaphoreType.DMA,
            pltpu.SemaphoreType.DMA,
            pltpu.SemaphoreType.DMA,
            pltpu.SemaphoreType.DMA,
        ],
    )(word_embeddings, ids3)


ROWS_BLK = 2048
POS_BLKS = S // ROWS_BLK


def _tc_epilogue(g_ref, p_ref, tt_tab_ref, tt_ref, gamma_ref, beta_ref, o_ref):
    x = g_ref[...] + p_ref[...]
    tt = tt_ref[0].astype(jnp.float32)
    row0 = tt_tab_ref[0:1, :]
    row1 = tt_tab_ref[1:2, :]
    x = x + row0 + tt * (row1 - row0)
    mean = jnp.mean(x, axis=-1, keepdims=True)
    d = x - mean
    var = jnp.mean(d * d, axis=-1, keepdims=True)
    o_ref[...] = d * lax.rsqrt(var + EPS) * gamma_ref[...] + beta_ref[...]


def _epilogue(gathered, position_embeddings, token_type_embeddings, tt_ids,
              ln_gamma, ln_beta):
    tt3 = tt_ids.reshape(N // ROWS_BLK, ROWS_BLK, 1)
    return pl.pallas_call(
        _tc_epilogue,
        grid=(POS_BLKS, B),
        in_specs=[
            pl.BlockSpec((ROWS_BLK, D), lambda p, b: (b * POS_BLKS + p, 0)),
            pl.BlockSpec((ROWS_BLK, D), lambda p, b: (p, 0)),
            pl.BlockSpec((2, D), lambda p, b: (0, 0)),
            pl.BlockSpec((1, ROWS_BLK, 1), lambda p, b: (b * POS_BLKS + p, 0, 0)),
            pl.BlockSpec((1, D), lambda p, b: (0, 0)),
            pl.BlockSpec((1, D), lambda p, b: (0, 0)),
        ],
        out_specs=pl.BlockSpec((ROWS_BLK, D), lambda p, b: (b * POS_BLKS + p, 0)),
        out_shape=jax.ShapeDtypeStruct((N, D), jnp.float32),
    )(gathered, position_embeddings, token_type_embeddings, tt3,
      ln_gamma.reshape(1, D), ln_beta.reshape(1, D))


def kernel(input_ids, token_type_ids, word_embeddings, position_embeddings,
           token_type_embeddings, ln_gamma, ln_beta):
    ids = input_ids.astype(jnp.int32)
    tt_ids = token_type_ids.astype(jnp.int32)
    g = _gather_rows(word_embeddings, ids.reshape(N))
    out = _epilogue(g, position_embeddings, token_type_embeddings,
                    tt_ids.reshape(N), ln_gamma, ln_beta)
    return out.reshape(B, S, D)


# speedup vs baseline: 1.6001x; 1.0021x over previous
import jax
import jax.numpy as jnp
from jax import lax
from jax.experimental import pallas as pl
from jax.experimental.pallas import tpu as pltpu
from jax.experimental.pallas import tpu_sc as plsc

B, S, D = 4, 2048, 768
N = B * S
EPS = 1e-12

_info = plsc.get_sparse_core_info()
NC, NS = _info.num_cores, _info.num_subcores
NW = NC * NS
TOK_PER_W = N // NW
CHUNK = 64
NCHUNK = TOK_PER_W // CHUNK


def _sc_gather(word_hbm, ids_hbm, out_hbm, idx_v, buf0, buf1, gs0, gs1, ss0,
               ss1):
    wid = lax.axis_index("s") * NC + lax.axis_index("c")
    base = wid * TOK_PER_W
    pltpu.sync_copy(ids_hbm.at[pl.ds(base, TOK_PER_W)], idx_v)
    bufs = (buf0, buf1)
    gsems = (gs0, gs1)
    ssems = (ss0, ss1)
    gathers = [None] * NCHUNK
    scatters = [None] * NCHUNK
    gathers[0] = pltpu.async_copy(
        word_hbm.at[idx_v.at[pl.ds(0, CHUNK)]], bufs[0], gsems[0])
    for j in range(NCHUNK):
        b = j % 2
        if j + 1 < NCHUNK:
            if j - 1 >= 0:
                scatters[j - 1].wait()
            gathers[j + 1] = pltpu.async_copy(
                word_hbm.at[idx_v.at[pl.ds((j + 1) * CHUNK, CHUNK)]],
                bufs[1 - b], gsems[1 - b])
        gathers[j].wait()
        scatters[j] = pltpu.async_copy(
            bufs[b], out_hbm.at[pl.ds(base + j * CHUNK, CHUNK)], ssems[b])
    scatters[NCHUNK - 2].wait()
    scatters[NCHUNK - 1].wait()


def _gather_rows(word_embeddings, ids):
    mesh = plsc.VectorSubcoreMesh(core_axis_name="c", subcore_axis_name="s")
    return pl.kernel(
        _sc_gather,
        mesh=mesh,
        out_type=jax.ShapeDtypeStruct((N, D), jnp.float32),
        scratch_types=[
            pltpu.VMEM((TOK_PER_W,), jnp.int32),
            pltpu.VMEM((CHUNK, D), jnp.float32),
            pltpu.VMEM((CHUNK, D), jnp.float32),
            pltpu.SemaphoreType.DMA,
            pltpu.SemaphoreType.DMA,
            pltpu.SemaphoreType.DMA,
            pltpu.SemaphoreType.DMA,
        ],
    )(word_embeddings, ids)


def _tc_epilogue(g_ref, p_ref, tt_tab_ref, tt_ref, gamma_ref, beta_ref, o_ref):
    x = g_ref[...] + p_ref[...]
    tt = tt_ref[0].astype(jnp.float32)
    row0 = tt_tab_ref[0:1, :]
    row1 = tt_tab_ref[1:2, :]
    x = x + row0 + tt * (row1 - row0)
    mean = jnp.mean(x, axis=-1, keepdims=True)
    d = x - mean
    var = jnp.mean(d * d, axis=-1, keepdims=True)
    o_ref[...] = d * lax.rsqrt(var + EPS) * gamma_ref[...] + beta_ref[...]


def _epilogue(gathered, position_embeddings, token_type_embeddings, tt_ids,
              ln_gamma, ln_beta):
    tt3 = tt_ids.reshape(B, S, 1)
    return pl.pallas_call(
        _tc_epilogue,
        grid=(B,),
        in_specs=[
            pl.BlockSpec((S, D), lambda b: (b, 0)),
            pl.BlockSpec((S, D), lambda b: (0, 0)),
            pl.BlockSpec((2, D), lambda b: (0, 0)),
            pl.BlockSpec((1, S, 1), lambda b: (b, 0, 0)),
            pl.BlockSpec((1, D), lambda b: (0, 0)),
            pl.BlockSpec((1, D), lambda b: (0, 0)),
        ],
        out_specs=pl.BlockSpec((S, D), lambda b: (b, 0)),
        out_shape=jax.ShapeDtypeStruct((N, D), jnp.float32),
    )(gathered, position_embeddings, token_type_embeddings, tt3,
      ln_gamma.reshape(1, D), ln_beta.reshape(1, D))


def kernel(input_ids, token_type_ids, word_embeddings, position_embeddings,
           token_type_embeddings, ln_gamma, ln_beta):
    ids = input_ids.astype(jnp.int32).reshape(N)
    tt_ids = token_type_ids.astype(jnp.int32)
    g = _gather_rows(word_embeddings, ids)
    out = _epilogue(g, position_embeddings, token_type_embeddings, tt_ids,
                    ln_gamma, ln_beta)
    return out.reshape(B, S, D)


# speedup vs baseline: 1.6217x; 1.0135x over previous
import jax
import jax.numpy as jnp
from jax import lax
from jax.experimental import pallas as pl
from jax.experimental.pallas import tpu as pltpu
from jax.experimental.pallas import tpu_sc as plsc

B, S, D = 4, 2048, 768
N = B * S
EPS = 1e-12

_info = plsc.get_sparse_core_info()
NC, NS = _info.num_cores, _info.num_subcores
NW = NC * NS
TOK_PER_W = N // NW
CHUNK = 32
NCHUNK = TOK_PER_W // CHUNK
NBUF = 4
AHEAD = 2


def _sc_gather(word_hbm, ids_hbm, out_hbm, idx_v, *rest):
    bufs = rest[:NBUF]
    gsems = rest[NBUF:2 * NBUF]
    ssems = rest[2 * NBUF:3 * NBUF]
    wid = lax.axis_index("s") * NC + lax.axis_index("c")
    base = wid * TOK_PER_W
    pltpu.sync_copy(ids_hbm.at[pl.ds(base, TOK_PER_W)], idx_v)

    def gather(k):
        return pltpu.async_copy(
            word_hbm.at[idx_v.at[pl.ds(k * CHUNK, CHUNK)]], bufs[k % NBUF],
            gsems[k % NBUF])

    gathers = [None] * NCHUNK
    scatters = [None] * NCHUNK
    waited = [False] * NCHUNK
    for k in range(min(AHEAD, NCHUNK)):
        gathers[k] = gather(k)
    for j in range(NCHUNK):
        k = j + AHEAD
        if k < NCHUNK:
            if k - NBUF >= 0:
                scatters[k - NBUF].wait()
                waited[k - NBUF] = True
            gathers[k] = gather(k)
        gathers[j].wait()
        scatters[j] = pltpu.async_copy(
            bufs[j % NBUF], out_hbm.at[pl.ds(base + j * CHUNK, CHUNK)],
            ssems[j % NBUF])
    for j in range(NCHUNK):
        if not waited[j]:
            scatters[j].wait()


def _gather_rows(word_embeddings, ids):
    mesh = plsc.VectorSubcoreMesh(core_axis_name="c", subcore_axis_name="s")
    return pl.kernel(
        _sc_gather,
        mesh=mesh,
        out_type=jax.ShapeDtypeStruct((N, D), jnp.float32),
        scratch_types=[pltpu.VMEM((TOK_PER_W,), jnp.int32)]
        + [pltpu.VMEM((CHUNK, D), jnp.float32) for _ in range(NBUF)]
        + [pltpu.SemaphoreType.DMA for _ in range(2 * NBUF)],
    )(word_embeddings, ids)


def _tc_epilogue(g_ref, p_ref, tt_tab_ref, tt_ref, gamma_ref, beta_ref, o_ref):
    x = g_ref[...] + p_ref[...]
    tt = tt_ref[0].astype(jnp.float32)
    row0 = tt_tab_ref[0:1, :]
    row1 = tt_tab_ref[1:2, :]
    x = x + row0 + tt * (row1 - row0)
    mean = jnp.mean(x, axis=-1, keepdims=True)
    d = x - mean
    var = jnp.mean(d * d, axis=-1, keepdims=True)
    o_ref[...] = d * lax.rsqrt(var + EPS) * gamma_ref[...] + beta_ref[...]


def _epilogue(gathered, position_embeddings, token_type_embeddings, tt_ids,
              ln_gamma, ln_beta):
    tt3 = tt_ids.reshape(B, S, 1)
    return pl.pallas_call(
        _tc_epilogue,
        grid=(B,),
        in_specs=[
            pl.BlockSpec((S, D), lambda b: (b, 0)),
            pl.BlockSpec((S, D), lambda b: (0, 0)),
            pl.BlockSpec((2, D), lambda b: (0, 0)),
            pl.BlockSpec((1, S, 1), lambda b: (b, 0, 0)),
            pl.BlockSpec((1, D), lambda b: (0, 0)),
            pl.BlockSpec((1, D), lambda b: (0, 0)),
        ],
        out_specs=pl.BlockSpec((S, D), lambda b: (b, 0)),
        out_shape=jax.ShapeDtypeStruct((N, D), jnp.float32),
    )(gathered, position_embeddings, token_type_embeddings, tt3,
      ln_gamma.reshape(1, D), ln_beta.reshape(1, D))


def kernel(input_ids, token_type_ids, word_embeddings, position_embeddings,
           token_type_embeddings, ln_gamma, ln_beta):
    ids = input_ids.astype(jnp.int32).reshape(N)
    tt_ids = token_type_ids.astype(jnp.int32)
    g = _gather_rows(word_embeddings, ids)
    out = _epilogue(g, position_embeddings, token_type_embeddings, tt_ids,
                    ln_gamma, ln_beta)
    return out.reshape(B, S, D)


# speedup vs baseline: 1.6411x; 1.0120x over previous
import jax
import jax.numpy as jnp
from jax import lax
from jax.experimental import pallas as pl
from jax.experimental.pallas import tpu as pltpu
from jax.experimental.pallas import tpu_sc as plsc

B, S, D = 4, 2048, 768
N = B * S
EPS = 1e-12

_info = plsc.get_sparse_core_info()
NC, NS = _info.num_cores, _info.num_subcores
NW = NC * NS
TOK_PER_W = N // NW
CHUNK = 32
NCHUNK = TOK_PER_W // CHUNK
NBUF = 4
AHEAD = 2


def _sc_gather(word_hbm, ids_hbm, out_hbm, idx_v, *rest):
    bufs = rest[:NBUF]
    gsems = rest[NBUF:2 * NBUF]
    ssems = rest[2 * NBUF:3 * NBUF]
    wid = lax.axis_index("s") * NC + lax.axis_index("c")
    base = wid * TOK_PER_W
    pltpu.sync_copy(ids_hbm.at[pl.ds(base, TOK_PER_W)], idx_v)

    def gather(k):
        return pltpu.async_copy(
            word_hbm.at[idx_v.at[pl.ds(k * CHUNK, CHUNK)]], bufs[k % NBUF],
            gsems[k % NBUF])

    gathers = [None] * NCHUNK
    scatters = [None] * NCHUNK
    waited = [False] * NCHUNK
    for k in range(min(AHEAD, NCHUNK)):
        gathers[k] = gather(k)
    for j in range(NCHUNK):
        k = j + AHEAD
        if k < NCHUNK:
            if k - NBUF >= 0:
                scatters[k - NBUF].wait()
                waited[k - NBUF] = True
            gathers[k] = gather(k)
        gathers[j].wait()
        scatters[j] = pltpu.async_copy(
            bufs[j % NBUF], out_hbm.at[pl.ds(base + j * CHUNK, CHUNK)],
            ssems[j % NBUF])
    for j in range(NCHUNK):
        if not waited[j]:
            scatters[j].wait()


def _gather_rows(word_embeddings, ids):
    mesh = plsc.VectorSubcoreMesh(core_axis_name="c", subcore_axis_name="s")
    return pl.kernel(
        _sc_gather,
        mesh=mesh,
        out_type=jax.ShapeDtypeStruct((N, D), jnp.float32),
        scratch_types=[pltpu.VMEM((TOK_PER_W,), jnp.int32)]
        + [pltpu.VMEM((CHUNK, D), jnp.float32) for _ in range(NBUF)]
        + [pltpu.SemaphoreType.DMA for _ in range(2 * NBUF)],
    )(word_embeddings, ids)


def _tc_epilogue(g_ref, p_ref, tt_tab_ref, tt_ref, gamma_ref, beta_ref, o_ref):
    x = g_ref[...] + p_ref[...]
    tt = tt_ref[0].astype(jnp.float32)
    row0 = tt_tab_ref[0:1, :]
    row1 = tt_tab_ref[1:2, :]
    x = x + row0 + tt * (row1 - row0)
    mean = jnp.mean(x, axis=-1, keepdims=True)
    d = x - mean
    var = jnp.mean(d * d, axis=-1, keepdims=True)
    o_ref[...] = d * lax.rsqrt(var + EPS) * gamma_ref[...] + beta_ref[...]


def _epilogue(gathered, position_embeddings, token_type_embeddings, tt_ids,
              ln_gamma, ln_beta):
    tt3 = tt_ids.reshape(B, S, 1).astype(jnp.int8)
    return pl.pallas_call(
        _tc_epilogue,
        grid=(B,),
        in_specs=[
            pl.BlockSpec((S, D), lambda b: (b, 0)),
            pl.BlockSpec((S, D), lambda b: (0, 0)),
            pl.BlockSpec((2, D), lambda b: (0, 0)),
            pl.BlockSpec((1, S, 1), lambda b: (b, 0, 0)),
            pl.BlockSpec((1, D), lambda b: (0, 0)),
            pl.BlockSpec((1, D), lambda b: (0, 0)),
        ],
        out_specs=pl.BlockSpec((S, D), lambda b: (b, 0)),
        out_shape=jax.ShapeDtypeStruct((N, D), jnp.float32),
        input_output_aliases={0: 0},
    )(gathered, position_embeddings, token_type_embeddings, tt3,
      ln_gamma.reshape(1, D), ln_beta.reshape(1, D))


def kernel(input_ids, token_type_ids, word_embeddings, position_embeddings,
           token_type_embeddings, ln_gamma, ln_beta):
    ids = input_ids.astype(jnp.int32).reshape(N)
    tt_ids = token_type_ids.astype(jnp.int32)
    g = _gather_rows(word_embeddings, ids)
    out = _epilogue(g, position_embeddings, token_type_embeddings, tt_ids,
                    ln_gamma, ln_beta)
    return out.reshape(B, S, D)


# speedup vs baseline: 1.6445x; 1.0020x over previous
import jax
import jax.numpy as jnp
from jax import lax
from jax.experimental import pallas as pl
from jax.experimental.pallas import tpu as pltpu
from jax.experimental.pallas import tpu_sc as plsc

B, S, D = 4, 2048, 768
N = B * S
EPS = 1e-12

_info = plsc.get_sparse_core_info()
NC, NS = _info.num_cores, _info.num_subcores
NW = NC * NS
TOK_PER_W = N // NW
CHUNK = 32
NCHUNK = TOK_PER_W // CHUNK
NBUF = 4
AHEAD = 2


def _sc_gather(word_hbm, ids_hbm, out_hbm, idx_v, *rest):
    bufs = rest[:NBUF]
    gsems = rest[NBUF:2 * NBUF]
    ssems = rest[2 * NBUF:3 * NBUF]
    wid = lax.axis_index("s") * NC + lax.axis_index("c")
    base = wid * TOK_PER_W
    w_per_row = S // TOK_PER_W
    pltpu.sync_copy(
        ids_hbm.at[wid // w_per_row,
                   pl.ds((wid % w_per_row) * TOK_PER_W, TOK_PER_W)], idx_v)

    def gather(k):
        return pltpu.async_copy(
            word_hbm.at[idx_v.at[pl.ds(k * CHUNK, CHUNK)]], bufs[k % NBUF],
            gsems[k % NBUF])

    gathers = [None] * NCHUNK
    scatters = [None] * NCHUNK
    waited = [False] * NCHUNK
    for k in range(min(AHEAD, NCHUNK)):
        gathers[k] = gather(k)
    for j in range(NCHUNK):
        k = j + AHEAD
        if k < NCHUNK:
            if k - NBUF >= 0:
                scatters[k - NBUF].wait()
                waited[k - NBUF] = True
            gathers[k] = gather(k)
        gathers[j].wait()
        scatters[j] = pltpu.async_copy(
            bufs[j % NBUF], out_hbm.at[pl.ds(base + j * CHUNK, CHUNK)],
            ssems[j % NBUF])
    for j in range(NCHUNK):
        if not waited[j]:
            scatters[j].wait()


def _gather_rows(word_embeddings, ids):
    mesh = plsc.VectorSubcoreMesh(core_axis_name="c", subcore_axis_name="s")
    return pl.kernel(
        _sc_gather,
        mesh=mesh,
        out_type=jax.ShapeDtypeStruct((N, D), jnp.float32),
        scratch_types=[pltpu.VMEM((TOK_PER_W,), jnp.int32)]
        + [pltpu.VMEM((CHUNK, D), jnp.float32) for _ in range(NBUF)]
        + [pltpu.SemaphoreType.DMA for _ in range(2 * NBUF)],
    )(word_embeddings, ids)


def _tc_epilogue(g_ref, p_ref, tt_tab_ref, tt_ref, gamma_ref, beta_ref, o_ref):
    x = g_ref[...] + p_ref[...]
    tt = tt_ref[0].astype(jnp.float32)
    row0 = tt_tab_ref[0:1, :]
    row1 = tt_tab_ref[1:2, :]
    x = x + row0 + tt * (row1 - row0)
    mean = jnp.mean(x, axis=-1, keepdims=True)
    d = x - mean
    var = jnp.mean(d * d, axis=-1, keepdims=True)
    o_ref[...] = d * lax.rsqrt(var + EPS) * gamma_ref[...] + beta_ref[...]


def _epilogue(gathered, position_embeddings, token_type_embeddings, tt_ids,
              ln_gamma, ln_beta):
    tt3 = tt_ids.reshape(B, S, 1).astype(jnp.int8)
    return pl.pallas_call(
        _tc_epilogue,
        grid=(B,),
        in_specs=[
            pl.BlockSpec((S, D), lambda b: (b, 0)),
            pl.BlockSpec((S, D), lambda b: (0, 0)),
            pl.BlockSpec((2, D), lambda b: (0, 0)),
            pl.BlockSpec((1, S, 1), lambda b: (b, 0, 0)),
            pl.BlockSpec((1, D), lambda b: (0, 0)),
            pl.BlockSpec((1, D), lambda b: (0, 0)),
        ],
        out_specs=pl.BlockSpec((S, D), lambda b: (b, 0)),
        out_shape=jax.ShapeDtypeStruct((N, D), jnp.float32),
        input_output_aliases={0: 0},
    )(gathered, position_embeddings, token_type_embeddings, tt3,
      ln_gamma.reshape(1, D), ln_beta.reshape(1, D))


def kernel(input_ids, token_type_ids, word_embeddings, position_embeddings,
           token_type_embeddings, ln_gamma, ln_beta):
    ids = input_ids.astype(jnp.int32)
    tt_ids = token_type_ids.astype(jnp.int32)
    g = _gather_rows(word_embeddings, ids)
    out = _epilogue(g, position_embeddings, token_type_embeddings, tt_ids,
                    ln_gamma, ln_beta)
    return out.reshape(B, S, D)


# speedup vs baseline: 1.6449x; 1.0003x over previous
import jax
import jax.numpy as jnp
from jax import lax
from jax.experimental import pallas as pl
from jax.experimental.pallas import tpu as pltpu
from jax.experimental.pallas import tpu_sc as plsc

B, S, D = 4, 2048, 768
N = B * S
EPS = 1e-12

_info = plsc.get_sparse_core_info()
NC, NS = _info.num_cores, _info.num_subcores
NW = NC * NS
TOK_PER_W = N // NW
CHUNK = 32
NCHUNK = TOK_PER_W // CHUNK
NBUF = 4
AHEAD = 2


def _sc_gather(word_hbm, ids_hbm, out_hbm, idx_v, *rest):
    bufs = rest[:NBUF]
    gsems = rest[NBUF:2 * NBUF]
    ssems = rest[2 * NBUF:3 * NBUF]
    wid = lax.axis_index("s") * NC + lax.axis_index("c")
    base = wid * TOK_PER_W
    w_per_row = S // TOK_PER_W
    pltpu.sync_copy(
        ids_hbm.at[wid // w_per_row,
                   pl.ds((wid % w_per_row) * TOK_PER_W, TOK_PER_W)], idx_v)

    def gather(k):
        return pltpu.async_copy(
            word_hbm.at[idx_v.at[pl.ds(k * CHUNK, CHUNK)]], bufs[k % NBUF],
            gsems[k % NBUF])

    gathers = [None] * NCHUNK
    scatters = [None] * NCHUNK
    waited = [False] * NCHUNK
    for k in range(min(AHEAD, NCHUNK)):
        gathers[k] = gather(k)
    for j in range(NCHUNK):
        k = j + AHEAD
        if k < NCHUNK:
            if k - NBUF >= 0:
                scatters[k - NBUF].wait()
                waited[k - NBUF] = True
            gathers[k] = gather(k)
        gathers[j].wait()
        scatters[j] = pltpu.async_copy(
            bufs[j % NBUF], out_hbm.at[pl.ds(base + j * CHUNK, CHUNK)],
            ssems[j % NBUF])
    for j in range(NCHUNK):
        if not waited[j]:
            scatters[j].wait()


def _gather_rows(word_embeddings, ids):
    mesh = plsc.VectorSubcoreMesh(core_axis_name="c", subcore_axis_name="s")
    return pl.kernel(
        _sc_gather,
        mesh=mesh,
        out_type=jax.ShapeDtypeStruct((N, D), jnp.float32),
        scratch_types=[pltpu.VMEM((TOK_PER_W,), jnp.int32)]
        + [pltpu.VMEM((CHUNK, D), jnp.float32) for _ in range(NBUF)]
        + [pltpu.SemaphoreType.DMA for _ in range(2 * NBUF)],
    )(word_embeddings, ids)


def _tc_epilogue(g_ref, p_ref, tt_tab_ref, tt_ref, gamma_ref, beta_ref, o_ref):
    x = g_ref[...] + p_ref[...]
    tt = tt_ref[0].astype(jnp.float32)
    row0 = tt_tab_ref[0:1, :]
    row1 = tt_tab_ref[1:2, :]
    x = x + row0 + tt * (row1 - row0)
    mean = jnp.mean(x, axis=-1, keepdims=True)
    d = x - mean
    var = jnp.mean(d * d, axis=-1, keepdims=True)
    o_ref[...] = (d * lax.rsqrt(var + EPS) * gamma_ref[...][None, :]
                  + beta_ref[...][None, :])


def _epilogue(gathered, position_embeddings, token_type_embeddings, tt_ids,
              ln_gamma, ln_beta):
    tt3 = tt_ids.reshape(B, S, 1).astype(jnp.int8)
    return pl.pallas_call(
        _tc_epilogue,
        grid=(B,),
        in_specs=[
            pl.BlockSpec((S, D), lambda b: (b, 0)),
            pl.BlockSpec((S, D), lambda b: (0, 0)),
            pl.BlockSpec((2, D), lambda b: (0, 0)),
            pl.BlockSpec((1, S, 1), lambda b: (b, 0, 0)),
            pl.BlockSpec((D,), lambda b: (0,)),
            pl.BlockSpec((D,), lambda b: (0,)),
        ],
        out_specs=pl.BlockSpec((S, D), lambda b: (b, 0)),
        out_shape=jax.ShapeDtypeStruct((N, D), jnp.float32),
        input_output_aliases={0: 0},
    )(gathered, position_embeddings, token_type_embeddings, tt3,
      ln_gamma, ln_beta)


def kernel(input_ids, token_type_ids, word_embeddings, position_embeddings,
           token_type_embeddings, ln_gamma, ln_beta):
    ids = input_ids.astype(jnp.int32)
    tt_ids = token_type_ids.astype(jnp.int32)
    g = _gather_rows(word_embeddings, ids)
    out = _epilogue(g, position_embeddings, token_type_embeddings, tt_ids,
                    ln_gamma, ln_beta)
    return out.reshape(B, S, D)
